# Initial kernel scaffold; baseline (speedup 1.0000x reference)
import functools

import jax
import jax.numpy as jnp
from jax import lax
from jax.experimental import pallas as pl
from jax.experimental.pallas import tpu as pltpu
from jax.experimental.pallas import tpu_sc as plsc

N = 10000
NPAD = 10240
E = 160000
EPAD = 163840
DIN = 128
DH = 512
DOUT = 128
G = 64

NC = 2
NS = 16
W = 128
RPT = NPAD // NS
C1 = DIN + 16

_mesh = plsc.VectorSubcoreMesh(core_axis_name="c", subcore_axis_name="s")


def _make_edge_split_agg(C):
    EPW = EPAD // (NC * NS)
    NWIN = EPW // W

    @functools.partial(
        pl.kernel,
        out_type=jax.ShapeDtypeStruct((NC * NPAD, C), jnp.float32),
        mesh=_mesh,
        scratch_types=[
            pltpu.VMEM((W,), jnp.int32),
            pltpu.VMEM((W,), jnp.int32),
            pltpu.VMEM((W, C), jnp.float32),
            pltpu.VMEM_SHARED((NPAD, C), jnp.float32),
            pltpu.SemaphoreType.DMA,
        ],
    )
    def agg(h_hbm, src_hbm, dst_hbm, zer_hbm, out_hbm, idx_s, idx_d, rows,
            acc, sem):
        c = lax.axis_index("c")
        s = lax.axis_index("s")
        w = s * NC + c
        r0 = s * RPT
        pltpu.sync_copy(zer_hbm.at[pl.ds(r0, RPT)], acc.at[pl.ds(r0, RPT)])
        plsc.subcore_barrier()

        def body(j, carry):
            base = pl.multiple_of(w * EPW + j * W, W)
            pltpu.sync_copy(src_hbm.at[pl.ds(base, W)], idx_s)
            pltpu.sync_copy(dst_hbm.at[pl.ds(base, W)], idx_d)
            pltpu.async_copy(h_hbm.at[idx_s], rows, sem).wait()
            pltpu.sync_copy(rows, acc.at[idx_d], add=True)
            return carry

        lax.fori_loop(0, NWIN, body, 0)
        plsc.subcore_barrier()
        pltpu.sync_copy(acc.at[pl.ds(r0, RPT)],
                        out_hbm.at[pl.ds(c * NPAD + r0, RPT)])

    return agg


_agg_l1 = _make_edge_split_agg(C1)
_agg_l3 = _make_edge_split_agg(DOUT)


def _make_chunk_agg():
    EPT = EPAD // NS
    NWIN = EPT // W

    @functools.partial(
        pl.kernel,
        out_type=[jax.ShapeDtypeStruct((NPAD, DIN), jnp.float32)] * 4,
        mesh=_mesh,
        scratch_types=[
            pltpu.VMEM((W,), jnp.int32),
            pltpu.VMEM((W,), jnp.int32),
            pltpu.VMEM((W, DIN), jnp.float32),
            pltpu.VMEM_SHARED((NPAD, DIN), jnp.float32),
            pltpu.SemaphoreType.DMA,
        ],
    )
    def agg4(h0, h1, h2, h3, src_hbm, dst_hbm, zer_hbm,
             o0, o1, o2, o3, idx_s, idx_d, rows, acc, sem):
        c = lax.axis_index("c")
        s = lax.axis_index("s")
        r0 = s * RPT
        hs = (h0, h1, h2, h3)
        os_ = (o0, o1, o2, o3)
        for chunk in range(4):
            h_hbm = hs[chunk]
            out_hbm = os_[chunk]

            @pl.when(c == (chunk % 2))
            def _process():
                pltpu.sync_copy(zer_hbm.at[pl.ds(r0, RPT)],
                                acc.at[pl.ds(r0, RPT)])
                plsc.subcore_barrier()

                def body(j, carry):
                    base = pl.multiple_of(s * EPT + j * W, W)
                    pltpu.sync_copy(src_hbm.at[pl.ds(base, W)], idx_s)
                    pltpu.sync_copy(dst_hbm.at[pl.ds(base, W)], idx_d)
                    pltpu.async_copy(h_hbm.at[idx_s], rows, sem).wait()
                    pltpu.sync_copy(rows, acc.at[idx_d], add=True)
                    return carry

                lax.fori_loop(0, NWIN, body, 0)
                plsc.subcore_barrier()
                pltpu.sync_copy(acc.at[pl.ds(r0, RPT)],
                                out_hbm.at[pl.ds(r0, RPT)])
                plsc.subcore_barrier()

    return agg4


_agg_l2 = _make_chunk_agg()

R = 256
NBLK = NPAD // R


def _l1_body(s1_ref, x_ref, wl_ref, bl_ref, wr_ref,
             h0_ref, h1_ref, h2_ref, h3_ref, rb_ref):
    ssum = s1_ref[0] + s1_ref[1]
    cnt = ssum[:, DIN:DIN + 1]
    recip = 1.0 / jnp.maximum(cnt, 1.0)
    aggv = ssum[:, :DIN] * recip
    h = (jnp.dot(aggv, wl_ref[...], preferred_element_type=jnp.float32)
         + bl_ref[...]
         + jnp.dot(x_ref[...], wr_ref[...], preferred_element_type=jnp.float32))
    h = jnp.maximum(h, 0.0)
    h0_ref[...] = h[:, 0:128]
    h1_ref[...] = h[:, 128:256]
    h2_ref[...] = h[:, 256:384]
    h3_ref[...] = h[:, 384:512]
    rb_ref[...] = jnp.broadcast_to(recip, (R, DIN))


def _tc_layer1(s1, x_pad, wl1, bl1, wr1):
    blk = lambda i: (i, 0)
    whole = lambda i: (0, 0)
    outs = jax.ShapeDtypeStruct((NPAD, DIN), jnp.float32)
    return pl.pallas_call(
        _l1_body,
        grid=(NBLK,),
        in_specs=[
            pl.BlockSpec((2, R, C1), lambda i: (0, i, 0)),
            pl.BlockSpec((R, DIN), blk),
            pl.BlockSpec((DIN, DH), whole),
            pl.BlockSpec((1, DH), whole),
            pl.BlockSpec((DIN, DH), whole),
        ],
        out_specs=[pl.BlockSpec((R, DIN), blk)] * 5,
        out_shape=[outs] * 5,
    )(s1, x_pad, wl1, bl1, wr1)


def _l2_body(s20, s21, s22, s23, h10, h11, h12, h13, rb_ref,
             wl2_ref, bl2_ref, wr2_ref, wl3_ref, wr3_ref,
             p3_ref, r3_ref):
    recip = rb_ref[:, 0:1]
    aggv = jnp.concatenate(
        [s20[...], s21[...], s22[...], s23[...]], axis=1) * recip
    h1 = jnp.concatenate([h10[...], h11[...], h12[...], h13[...]], axis=1)
    h2 = (jnp.dot(aggv, wl2_ref[...], preferred_element_type=jnp.float32)
          + bl2_ref[...]
          + jnp.dot(h1, wr2_ref[...], preferred_element_type=jnp.float32))
    h2 = jnp.maximum(h2, 0.0)
    p3_ref[...] = jnp.dot(h2, wl3_ref[...], preferred_element_type=jnp.float32)
    r3_ref[...] = jnp.dot(h2, wr3_ref[...], preferred_element_type=jnp.float32)


def _tc_layer2(s2s, h1s, recipb, wl2, bl2, wr2, wl3, wr3):
    blk = lambda i: (i, 0)
    whole = lambda i: (0, 0)
    outs = jax.ShapeDtypeStruct((NPAD, DOUT), jnp.float32)
    return pl.pallas_call(
        _l2_body,
        grid=(NBLK,),
        in_specs=(
            [pl.BlockSpec((R, DIN), blk)] * 8
            + [pl.BlockSpec((R, DIN), blk)]
            + [pl.BlockSpec((DH, DH), whole),
               pl.BlockSpec((1, DH), whole),
               pl.BlockSpec((DH, DH), whole),
               pl.BlockSpec((DH, DOUT), whole),
               pl.BlockSpec((DH, DOUT), whole)]
        ),
        out_specs=[pl.BlockSpec((R, DOUT), blk)] * 2,
        out_shape=[outs] * 2,
    )(*s2s, *h1s, recipb, wl2, bl2, wr2, wl3, wr3)


def _final_body(s3_ref, rb_ref, r3_ref, b_ref, bl3_ref, g_ref, be_ref,
                out_ref, psum, csum):
    i = pl.program_id(0)

    @pl.when(i == 0)
    def _init():
        psum[...] = jnp.zeros((G, DOUT), jnp.float32)
        csum[...] = jnp.zeros((G, 1), jnp.float32)

    ssum = s3_ref[0] + s3_ref[1]
    out3 = ssum * rb_ref[:, 0:1] + r3_ref[...] + bl3_ref[...]
    bb = b_ref[0]
    gids = lax.broadcasted_iota(jnp.float32, (G, R), 0)
    onehot = jnp.where(gids == bb, 1.0, 0.0)
    psum[...] += jnp.dot(onehot, out3, preferred_element_type=jnp.float32)
    csum[...] += jnp.sum(onehot, axis=1, keepdims=True)

    @pl.when(i == NBLK - 1)
    def _finish():
        pooled = psum[...] / jnp.maximum(csum[...], 1.0)
        mu = jnp.mean(pooled, axis=1, keepdims=True)
        var = jnp.mean((pooled - mu) ** 2, axis=1, keepdims=True)
        out_ref[...] = ((pooled - mu) * lax.rsqrt(var + 1e-5)
                        * g_ref[...] + be_ref[...])


def _tc_final(s3, recipb, r3, batchf, bl3, ln_g, ln_b):
    blk = lambda i: (i, 0)
    whole = lambda i: (0, 0)
    return pl.pallas_call(
        _final_body,
        grid=(NBLK,),
        in_specs=[
            pl.BlockSpec((2, R, DOUT), lambda i: (0, i, 0)),
            pl.BlockSpec((R, DIN), blk),
            pl.BlockSpec((R, DOUT), blk),
            pl.BlockSpec((1, 1, R), lambda i: (i, 0, 0)),
            pl.BlockSpec((1, DOUT), whole),
            pl.BlockSpec((1, DOUT), whole),
            pl.BlockSpec((1, DOUT), whole),
        ],
        out_specs=pl.BlockSpec((G, DOUT), whole),
        out_shape=jax.ShapeDtypeStruct((G, DOUT), jnp.float32),
        scratch_shapes=[
            pltpu.VMEM((G, DOUT), jnp.float32),
            pltpu.VMEM((G, 1), jnp.float32),
        ],
    )(s3, recipb, r3, batchf, bl3, ln_g, ln_b)


def kernel(x, edge_index, batch, Wl1, bl1, Wr1, Wl2, bl2, Wr2,
           Wl3, bl3, Wr3, ln_g, ln_b):
    f32 = jnp.float32
    src = edge_index[0]
    dst = edge_index[1]
    padidx = (N + (jnp.arange(EPAD - E, dtype=jnp.int32) % (NPAD - N)))
    srcp = jnp.concatenate([src, padidx])
    dstp = jnp.concatenate([dst, padidx])

    xa = jnp.concatenate(
        [x, jnp.ones((N, 1), f32), jnp.zeros((N, C1 - DIN - 1), f32)], axis=1)
    xa = jnp.concatenate([xa, jnp.zeros((NPAD - N, C1), f32)], axis=0)
    x_pad = jnp.concatenate([x, jnp.zeros((NPAD - N, DIN), f32)], axis=0)

    zer1 = jnp.zeros((NPAD, C1), f32)
    zer = jnp.zeros((NPAD, DIN), f32)

    s1 = _agg_l1(xa, srcp, dstp, zer1).reshape(2, NPAD, C1)
    h1s_and_recip = _tc_layer1(s1, x_pad, Wl1, bl1.reshape(1, DH), Wr1)
    h1s, recipb = h1s_and_recip[:4], h1s_and_recip[4]

    s2s = _agg_l2(*h1s, srcp, dstp, zer)
    p3, r3 = _tc_layer2(s2s, h1s, recipb, Wl2, bl2.reshape(1, DH), Wr2,
                        Wl3, Wr3)

    s3 = _agg_l3(p3, srcp, dstp, zer).reshape(2, NPAD, DOUT)
    batchf = jnp.concatenate(
        [batch.astype(f32), jnp.full((NPAD - N,), float(G), f32)]
    ).reshape(NBLK, 1, R)
    out = _tc_final(s3, recipb, r3, batchf, bl3.reshape(1, DOUT),
                    ln_g.reshape(1, DOUT), ln_b.reshape(1, DOUT))
    return out


# speedup vs baseline: 4.7995x; 4.7995x over previous
import functools

import jax
import jax.numpy as jnp
from jax import lax
from jax.experimental import pallas as pl
from jax.experimental.pallas import tpu as pltpu
from jax.experimental.pallas import tpu_sc as plsc

N = 10000
NPAD = 10240
E = 160000
EPAD = 163840
DIN = 128
DH = 512
DOUT = 128
G = 64

NC = 2
NS = 16
W = 128
RPT = NPAD // NS
C1 = DIN + 16

_mesh = plsc.VectorSubcoreMesh(core_axis_name="c", subcore_axis_name="s")


def _make_edge_split_agg(C):
    EPW = EPAD // (NC * NS)
    NWIN = EPW // W

    @functools.partial(
        pl.kernel,
        out_type=jax.ShapeDtypeStruct((NC * NPAD, C), jnp.float32),
        mesh=_mesh,
        scratch_types=[
            pltpu.VMEM((W,), jnp.int32),
            pltpu.VMEM((W,), jnp.int32),
            pltpu.VMEM((W, C), jnp.float32),
            pltpu.VMEM_SHARED((NPAD, C), jnp.float32),
            pltpu.SemaphoreType.DMA,
        ],
        compiler_params=pltpu.CompilerParams(use_tc_tiling_on_sc=False),
    )
    def agg(h_hbm, src_hbm, dst_hbm, zer_hbm, out_hbm, idx_s, idx_d, rows,
            acc, sem):
        c = lax.axis_index("c")
        s = lax.axis_index("s")
        w = s * NC + c
        r0 = s * RPT
        pltpu.sync_copy(zer_hbm.at[pl.ds(r0, RPT)], acc.at[pl.ds(r0, RPT)])
        plsc.subcore_barrier()

        def body(j, carry):
            base = pl.multiple_of(w * EPW + j * W, W)
            pltpu.sync_copy(src_hbm.at[pl.ds(base, W)], idx_s)
            pltpu.sync_copy(dst_hbm.at[pl.ds(base, W)], idx_d)
            pltpu.async_copy(h_hbm.at[idx_s], rows, sem).wait()
            pltpu.sync_copy(rows, acc.at[idx_d], add=True)
            return carry

        lax.fori_loop(0, NWIN, body, 0)
        plsc.subcore_barrier()
        pltpu.sync_copy(acc.at[pl.ds(r0, RPT)],
                        out_hbm.at[pl.ds(c * NPAD + r0, RPT)])

    return agg


_agg_l1 = _make_edge_split_agg(C1)
_agg_l3 = _make_edge_split_agg(DOUT)


def _make_chunk_agg():
    EPT = EPAD // NS
    NWIN = EPT // W

    @functools.partial(
        pl.kernel,
        out_type=[jax.ShapeDtypeStruct((NPAD, DIN), jnp.float32)] * 4,
        mesh=_mesh,
        scratch_types=[
            pltpu.VMEM((W,), jnp.int32),
            pltpu.VMEM((W,), jnp.int32),
            pltpu.VMEM((W, DIN), jnp.float32),
            pltpu.VMEM_SHARED((NPAD, DIN), jnp.float32),
            pltpu.SemaphoreType.DMA,
        ],
    )
    def agg4(h0, h1, h2, h3, src_hbm, dst_hbm, zer_hbm,
             o0, o1, o2, o3, idx_s, idx_d, rows, acc, sem):
        c = lax.axis_index("c")
        s = lax.axis_index("s")
        r0 = s * RPT
        hs = (h0, h1, h2, h3)
        os_ = (o0, o1, o2, o3)
        for chunk in range(4):
            h_hbm = hs[chunk]
            out_hbm = os_[chunk]

            @pl.when(c == (chunk % 2))
            def _process():
                pltpu.sync_copy(zer_hbm.at[pl.ds(r0, RPT)],
                                acc.at[pl.ds(r0, RPT)])
                plsc.subcore_barrier()

                def body(j, carry):
                    base = pl.multiple_of(s * EPT + j * W, W)
                    pltpu.sync_copy(src_hbm.at[pl.ds(base, W)], idx_s)
                    pltpu.sync_copy(dst_hbm.at[pl.ds(base, W)], idx_d)
                    pltpu.async_copy(h_hbm.at[idx_s], rows, sem).wait()
                    pltpu.sync_copy(rows, acc.at[idx_d], add=True)
                    return carry

                lax.fori_loop(0, NWIN, body, 0)
                plsc.subcore_barrier()
                pltpu.sync_copy(acc.at[pl.ds(r0, RPT)],
                                out_hbm.at[pl.ds(r0, RPT)])
                plsc.subcore_barrier()

    return agg4


_agg_l2 = _make_chunk_agg()

R = 256
NBLK = NPAD // R


def _l1_body(s1_ref, x_ref, wl_ref, bl_ref, wr_ref,
             h0_ref, h1_ref, h2_ref, h3_ref, rb_ref):
    ssum = s1_ref[0] + s1_ref[1]
    cnt = ssum[:, DIN:DIN + 1]
    recip = 1.0 / jnp.maximum(cnt, 1.0)
    aggv = ssum[:, :DIN] * recip
    h = (jnp.dot(aggv, wl_ref[...], preferred_element_type=jnp.float32)
         + bl_ref[...]
         + jnp.dot(x_ref[...], wr_ref[...], preferred_element_type=jnp.float32))
    h = jnp.maximum(h, 0.0)
    h0_ref[...] = h[:, 0:128]
    h1_ref[...] = h[:, 128:256]
    h2_ref[...] = h[:, 256:384]
    h3_ref[...] = h[:, 384:512]
    rb_ref[...] = jnp.broadcast_to(recip, (R, DIN))


def _tc_layer1(s1, x_pad, wl1, bl1, wr1):
    blk = lambda i: (i, 0)
    whole = lambda i: (0, 0)
    outs = jax.ShapeDtypeStruct((NPAD, DIN), jnp.float32)
    return pl.pallas_call(
        _l1_body,
        grid=(NBLK,),
        in_specs=[
            pl.BlockSpec((2, R, C1), lambda i: (0, i, 0)),
            pl.BlockSpec((R, DIN), blk),
            pl.BlockSpec((DIN, DH), whole),
            pl.BlockSpec((1, DH), whole),
            pl.BlockSpec((DIN, DH), whole),
        ],
        out_specs=[pl.BlockSpec((R, DIN), blk)] * 5,
        out_shape=[outs] * 5,
    )(s1, x_pad, wl1, bl1, wr1)


def _l2_body(s20, s21, s22, s23, h10, h11, h12, h13, rb_ref,
             wl2_ref, bl2_ref, wr2_ref, wl3_ref, wr3_ref,
             p3_ref, r3_ref):
    recip = rb_ref[:, 0:1]
    aggv = jnp.concatenate(
        [s20[...], s21[...], s22[...], s23[...]], axis=1) * recip
    h1 = jnp.concatenate([h10[...], h11[...], h12[...], h13[...]], axis=1)
    h2 = (jnp.dot(aggv, wl2_ref[...], preferred_element_type=jnp.float32)
          + bl2_ref[...]
          + jnp.dot(h1, wr2_ref[...], preferred_element_type=jnp.float32))
    h2 = jnp.maximum(h2, 0.0)
    p3_ref[...] = jnp.dot(h2, wl3_ref[...], preferred_element_type=jnp.float32)
    r3_ref[...] = jnp.dot(h2, wr3_ref[...], preferred_element_type=jnp.float32)


def _tc_layer2(s2s, h1s, recipb, wl2, bl2, wr2, wl3, wr3):
    blk = lambda i: (i, 0)
    whole = lambda i: (0, 0)
    outs = jax.ShapeDtypeStruct((NPAD, DOUT), jnp.float32)
    return pl.pallas_call(
        _l2_body,
        grid=(NBLK,),
        in_specs=(
            [pl.BlockSpec((R, DIN), blk)] * 8
            + [pl.BlockSpec((R, DIN), blk)]
            + [pl.BlockSpec((DH, DH), whole),
               pl.BlockSpec((1, DH), whole),
               pl.BlockSpec((DH, DH), whole),
               pl.BlockSpec((DH, DOUT), whole),
               pl.BlockSpec((DH, DOUT), whole)]
        ),
        out_specs=[pl.BlockSpec((R, DOUT), blk)] * 2,
        out_shape=[outs] * 2,
    )(*s2s, *h1s, recipb, wl2, bl2, wr2, wl3, wr3)


def _final_body(s3_ref, rb_ref, r3_ref, b_ref, bl3_ref, g_ref, be_ref,
                out_ref, psum, csum):
    i = pl.program_id(0)

    @pl.when(i == 0)
    def _init():
        psum[...] = jnp.zeros((G, DOUT), jnp.float32)
        csum[...] = jnp.zeros((G, 1), jnp.float32)

    ssum = s3_ref[0] + s3_ref[1]
    out3 = ssum * rb_ref[:, 0:1] + r3_ref[...] + bl3_ref[...]
    bb = b_ref[0]
    gids = lax.broadcasted_iota(jnp.int32, (G, R), 0).astype(jnp.float32)
    onehot = jnp.where(gids == bb, 1.0, 0.0)
    psum[...] += jnp.dot(onehot, out3, preferred_element_type=jnp.float32)
    csum[...] += jnp.sum(onehot, axis=1, keepdims=True)

    @pl.when(i == NBLK - 1)
    def _finish():
        pooled = psum[...] / jnp.maximum(csum[...], 1.0)
        mu = jnp.mean(pooled, axis=1, keepdims=True)
        var = jnp.mean((pooled - mu) ** 2, axis=1, keepdims=True)
        out_ref[...] = ((pooled - mu) * lax.rsqrt(var + 1e-5)
                        * g_ref[...] + be_ref[...])


def _tc_final(s3, recipb, r3, batchf, bl3, ln_g, ln_b):
    blk = lambda i: (i, 0)
    whole = lambda i: (0, 0)
    return pl.pallas_call(
        _final_body,
        grid=(NBLK,),
        in_specs=[
            pl.BlockSpec((2, R, DOUT), lambda i: (0, i, 0)),
            pl.BlockSpec((R, DIN), blk),
            pl.BlockSpec((R, DOUT), blk),
            pl.BlockSpec((1, 1, R), lambda i: (i, 0, 0)),
            pl.BlockSpec((1, DOUT), whole),
            pl.BlockSpec((1, DOUT), whole),
            pl.BlockSpec((1, DOUT), whole),
        ],
        out_specs=pl.BlockSpec((G, DOUT), whole),
        out_shape=jax.ShapeDtypeStruct((G, DOUT), jnp.float32),
        scratch_shapes=[
            pltpu.VMEM((G, DOUT), jnp.float32),
            pltpu.VMEM((G, 1), jnp.float32),
        ],
    )(s3, recipb, r3, batchf, bl3, ln_g, ln_b)


def kernel(x, edge_index, batch, Wl1, bl1, Wr1, Wl2, bl2, Wr2,
           Wl3, bl3, Wr3, ln_g, ln_b):
    f32 = jnp.float32
    src = edge_index[0]
    dst = edge_index[1]
    padidx = (N + (jnp.arange(EPAD - E, dtype=jnp.int32) % (NPAD - N)))
    srcp = jnp.concatenate([src, padidx])
    dstp = jnp.concatenate([dst, padidx])

    xa = jnp.concatenate(
        [x, jnp.ones((N, 1), f32), jnp.zeros((N, C1 - DIN - 1), f32)], axis=1)
    xa = jnp.concatenate([xa, jnp.zeros((NPAD - N, C1), f32)], axis=0)
    x_pad = jnp.concatenate([x, jnp.zeros((NPAD - N, DIN), f32)], axis=0)

    zer1 = jnp.zeros((NPAD, C1), f32)
    zer = jnp.zeros((NPAD, DIN), f32)

    s1 = _agg_l1(xa, srcp, dstp, zer1).reshape(2, NPAD, C1)
    h1s_and_recip = _tc_layer1(s1, x_pad, Wl1, bl1.reshape(1, DH), Wr1)
    h1s, recipb = h1s_and_recip[:4], h1s_and_recip[4]

    s2s = _agg_l2(*h1s, srcp, dstp, zer)
    p3, r3 = _tc_layer2(s2s, h1s, recipb, Wl2, bl2.reshape(1, DH), Wr2,
                        Wl3, Wr3)

    s3 = _agg_l3(p3, srcp, dstp, zer).reshape(2, NPAD, DOUT)
    batchf = jnp.concatenate(
        [batch.astype(f32), jnp.full((NPAD - N,), float(G), f32)]
    ).reshape(NBLK, 1, R)
    out = _tc_final(s3, recipb, r3, batchf, bl3.reshape(1, DOUT),
                    ln_g.reshape(1, DOUT), ln_b.reshape(1, DOUT))
    return out


# speedup vs baseline: 7.3807x; 1.5378x over previous
import functools

import jax
import jax.numpy as jnp
from jax import lax
from jax.experimental import pallas as pl
from jax.experimental.pallas import tpu as pltpu
from jax.experimental.pallas import tpu_sc as plsc

N = 10000
NPAD = 10240
E = 160000
EPAD = 163840
DIN = 128
DH = 512
DOUT = 128
G = 64

NC = 2
NS = 16
W = 128
RPT = NPAD // NS
C1 = DIN + 16

_mesh = plsc.VectorSubcoreMesh(core_axis_name="c", subcore_axis_name="s")


def _edge_loop_db(h_hbm, idxs, rows_a, rows_b, acc, gsa, gsb, nwin, dget):
    pltpu.async_copy(h_hbm.at[idxs.at[0]], rows_a, gsa)

    def body(k, carry):
        j0 = 2 * k
        pltpu.make_async_copy(h_hbm.at[idxs.at[j0]], rows_a, gsa).wait()
        db = pltpu.async_copy(h_hbm.at[idxs.at[j0 + 1]], rows_b, gsb)
        pltpu.sync_copy(rows_a, acc.at[dget(j0, 0)], add=True)
        db.wait()

        @pl.when(j0 + 2 < nwin)
        def _issue_a():
            pltpu.async_copy(h_hbm.at[idxs.at[j0 + 2]], rows_a, gsa)

        pltpu.sync_copy(rows_b, acc.at[dget(j0 + 1, 1)], add=True)
        return carry

    lax.fori_loop(0, nwin // 2, body, 0)


def _edge_loop_single(h_hbm, idxs, idxd, rows, acc, gsa, nwin):
    def body(j, carry):
        pltpu.async_copy(h_hbm.at[idxs.at[j]], rows, gsa).wait()
        pltpu.sync_copy(rows, acc.at[idxd.at[j]], add=True)
        return carry

    lax.fori_loop(0, nwin, body, 0)


def _make_edge_split_agg(C):
    EPW = EPAD // (NC * NS)
    NWIN = EPW // W
    double = C == DIN

    rows_scratch = [pltpu.VMEM((W, C), jnp.float32)] * (2 if double else 1)
    sem_scratch = [pltpu.SemaphoreType.DMA] * (2 if double else 1)

    @functools.partial(
        pl.kernel,
        out_type=jax.ShapeDtypeStruct((NC * NPAD, C), jnp.float32),
        mesh=_mesh,
        scratch_types=[
            pltpu.VMEM((NWIN, W), jnp.int32),
            pltpu.VMEM((NWIN, W), jnp.int32),
            *rows_scratch,
            pltpu.VMEM_SHARED((NPAD, C), jnp.float32),
            *sem_scratch,
        ],
        compiler_params=pltpu.CompilerParams(use_tc_tiling_on_sc=False),
    )
    def agg(h_hbm, src_hbm, dst_hbm, zer_hbm, out_hbm, idxs, idxd,
            *rest):
        if double:
            rows_a, rows_b, acc, gsa, gsb = rest
        else:
            rows_a, acc, gsa = rest
        c = lax.axis_index("c")
        s = lax.axis_index("s")
        w = s * NC + c
        r0 = s * RPT
        pltpu.sync_copy(zer_hbm.at[pl.ds(r0, RPT)], acc.at[pl.ds(r0, RPT)])
        pltpu.sync_copy(src_hbm.at[pl.ds(w * NWIN, NWIN)], idxs)
        pltpu.sync_copy(dst_hbm.at[pl.ds(w * NWIN, NWIN)], idxd)
        plsc.subcore_barrier()
        if double:
            _edge_loop_db(h_hbm, idxs, rows_a, rows_b, acc, gsa, gsb,
                          NWIN, lambda j, p: idxd.at[j])
        else:
            _edge_loop_single(h_hbm, idxs, idxd, rows_a, acc, gsa, NWIN)
        plsc.subcore_barrier()
        pltpu.sync_copy(acc.at[pl.ds(r0, RPT)],
                        out_hbm.at[pl.ds(c * NPAD + r0, RPT)])

    return agg


_agg_l1 = _make_edge_split_agg(C1)
_agg_l3 = _make_edge_split_agg(DOUT)


def _make_chunk_agg():
    EPT = EPAD // NS
    NWIN = EPT // W

    @functools.partial(
        pl.kernel,
        out_type=[jax.ShapeDtypeStruct((NPAD, DIN), jnp.float32)] * 4,
        mesh=_mesh,
        scratch_types=[
            pltpu.VMEM((NWIN, W), jnp.int32),
            pltpu.VMEM((1, W), jnp.int32),
            pltpu.VMEM((1, W), jnp.int32),
            pltpu.VMEM((W, DIN), jnp.float32),
            pltpu.VMEM((W, DIN), jnp.float32),
            pltpu.VMEM_SHARED((NPAD, DIN), jnp.float32),
            pltpu.SemaphoreType.DMA,
            pltpu.SemaphoreType.DMA,
        ],
        compiler_params=pltpu.CompilerParams(use_tc_tiling_on_sc=False),
    )
    def agg4(h0, h1, h2, h3, src_hbm, dst_hbm, zer_hbm,
             o0, o1, o2, o3, idxs, idxd_a, idxd_b, rows_a, rows_b,
             acc, gsa, gsb):
        c = lax.axis_index("c")
        s = lax.axis_index("s")
        r0 = s * RPT
        hs = (h0, h1, h2, h3)
        os_ = (o0, o1, o2, o3)
        dbufs = (idxd_a, idxd_b)
        pltpu.sync_copy(src_hbm.at[pl.ds(s * NWIN, NWIN)], idxs)

        def dget(j, p):
            buf = dbufs[p]
            pltpu.sync_copy(dst_hbm.at[pl.ds(s * NWIN + j, 1)], buf)
            return buf.at[0]

        for chunk in range(4):
            h_hbm = hs[chunk]
            out_hbm = os_[chunk]

            @pl.when(c == (chunk % 2))
            def _process():
                pltpu.sync_copy(zer_hbm.at[pl.ds(r0, RPT)],
                                acc.at[pl.ds(r0, RPT)])
                plsc.subcore_barrier()
                _edge_loop_db(h_hbm, idxs, rows_a, rows_b, acc,
                              gsa, gsb, NWIN, dget)
                plsc.subcore_barrier()
                pltpu.sync_copy(acc.at[pl.ds(r0, RPT)],
                                out_hbm.at[pl.ds(r0, RPT)])
                plsc.subcore_barrier()

    return agg4


_agg_l2 = _make_chunk_agg()

R = 256
NBLK = NPAD // R


def _l1_body(s1_ref, x_ref, wl_ref, bl_ref, wr_ref,
             h0_ref, h1_ref, h2_ref, h3_ref, rb_ref):
    ssum = s1_ref[0] + s1_ref[1]
    cnt = ssum[:, DIN:DIN + 1]
    recip = 1.0 / jnp.maximum(cnt, 1.0)
    aggv = ssum[:, :DIN] * recip
    h = (jnp.dot(aggv, wl_ref[...], preferred_element_type=jnp.float32)
         + bl_ref[...]
         + jnp.dot(x_ref[...], wr_ref[...], preferred_element_type=jnp.float32))
    h = jnp.maximum(h, 0.0)
    h0_ref[...] = h[:, 0:128]
    h1_ref[...] = h[:, 128:256]
    h2_ref[...] = h[:, 256:384]
    h3_ref[...] = h[:, 384:512]
    rb_ref[...] = jnp.broadcast_to(recip, (R, DIN))


def _tc_layer1(s1, x_pad, wl1, bl1, wr1):
    blk = lambda i: (i, 0)
    whole = lambda i: (0, 0)
    outs = jax.ShapeDtypeStruct((NPAD, DIN), jnp.float32)
    return pl.pallas_call(
        _l1_body,
        grid=(NBLK,),
        in_specs=[
            pl.BlockSpec((2, R, C1), lambda i: (0, i, 0)),
            pl.BlockSpec((R, DIN), blk),
            pl.BlockSpec((DIN, DH), whole),
            pl.BlockSpec((1, DH), whole),
            pl.BlockSpec((DIN, DH), whole),
        ],
        out_specs=[pl.BlockSpec((R, DIN), blk)] * 5,
        out_shape=[outs] * 5,
    )(s1, x_pad, wl1, bl1, wr1)


def _l2_body(s20, s21, s22, s23, h10, h11, h12, h13, rb_ref,
             wl2_ref, bl2_ref, wr2_ref, wl3_ref, wr3_ref,
             p3_ref, r3_ref):
    recip = rb_ref[:, 0:1]
    aggv = jnp.concatenate(
        [s20[...], s21[...], s22[...], s23[...]], axis=1) * recip
    h1 = jnp.concatenate([h10[...], h11[...], h12[...], h13[...]], axis=1)
    h2 = (jnp.dot(aggv, wl2_ref[...], preferred_element_type=jnp.float32)
          + bl2_ref[...]
          + jnp.dot(h1, wr2_ref[...], preferred_element_type=jnp.float32))
    h2 = jnp.maximum(h2, 0.0)
    p3_ref[...] = jnp.dot(h2, wl3_ref[...], preferred_element_type=jnp.float32)
    r3_ref[...] = jnp.dot(h2, wr3_ref[...], preferred_element_type=jnp.float32)


def _tc_layer2(s2s, h1s, recipb, wl2, bl2, wr2, wl3, wr3):
    blk = lambda i: (i, 0)
    whole = lambda i: (0, 0)
    outs = jax.ShapeDtypeStruct((NPAD, DOUT), jnp.float32)
    return pl.pallas_call(
        _l2_body,
        grid=(NBLK,),
        in_specs=(
            [pl.BlockSpec((R, DIN), blk)] * 8
            + [pl.BlockSpec((R, DIN), blk)]
            + [pl.BlockSpec((DH, DH), whole),
               pl.BlockSpec((1, DH), whole),
               pl.BlockSpec((DH, DH), whole),
               pl.BlockSpec((DH, DOUT), whole),
               pl.BlockSpec((DH, DOUT), whole)]
        ),
        out_specs=[pl.BlockSpec((R, DOUT), blk)] * 2,
        out_shape=[outs] * 2,
    )(*s2s, *h1s, recipb, wl2, bl2, wr2, wl3, wr3)


def _final_body(s3_ref, rb_ref, r3_ref, b_ref, bl3_ref, g_ref, be_ref,
                out_ref, psum, csum):
    i = pl.program_id(0)

    @pl.when(i == 0)
    def _init():
        psum[...] = jnp.zeros((G, DOUT), jnp.float32)
        csum[...] = jnp.zeros((G, 1), jnp.float32)

    ssum = s3_ref[0] + s3_ref[1]
    out3 = ssum * rb_ref[:, 0:1] + r3_ref[...] + bl3_ref[...]
    bb = b_ref[0]
    gids = lax.broadcasted_iota(jnp.int32, (G, R), 0).astype(jnp.float32)
    onehot = jnp.where(gids == bb, 1.0, 0.0)
    psum[...] += jnp.dot(onehot, out3, preferred_element_type=jnp.float32)
    csum[...] += jnp.sum(onehot, axis=1, keepdims=True)

    @pl.when(i == NBLK - 1)
    def _finish():
        pooled = psum[...] / jnp.maximum(csum[...], 1.0)
        mu = jnp.mean(pooled, axis=1, keepdims=True)
        var = jnp.mean((pooled - mu) ** 2, axis=1, keepdims=True)
        out_ref[...] = ((pooled - mu) * lax.rsqrt(var + 1e-5)
                        * g_ref[...] + be_ref[...])


def _tc_final(s3, recipb, r3, batchf, bl3, ln_g, ln_b):
    blk = lambda i: (i, 0)
    whole = lambda i: (0, 0)
    return pl.pallas_call(
        _final_body,
        grid=(NBLK,),
        in_specs=[
            pl.BlockSpec((2, R, DOUT), lambda i: (0, i, 0)),
            pl.BlockSpec((R, DIN), blk),
            pl.BlockSpec((R, DOUT), blk),
            pl.BlockSpec((1, 1, R), lambda i: (i, 0, 0)),
            pl.BlockSpec((1, DOUT), whole),
            pl.BlockSpec((1, DOUT), whole),
            pl.BlockSpec((1, DOUT), whole),
        ],
        out_specs=pl.BlockSpec((G, DOUT), whole),
        out_shape=jax.ShapeDtypeStruct((G, DOUT), jnp.float32),
        scratch_shapes=[
            pltpu.VMEM((G, DOUT), jnp.float32),
            pltpu.VMEM((G, 1), jnp.float32),
        ],
    )(s3, recipb, r3, batchf, bl3, ln_g, ln_b)


def kernel(x, edge_index, batch, Wl1, bl1, Wr1, Wl2, bl2, Wr2,
           Wl3, bl3, Wr3, ln_g, ln_b):
    f32 = jnp.float32
    src = edge_index[0]
    dst = edge_index[1]
    padidx = (N + (jnp.arange(EPAD - E, dtype=jnp.int32) % (NPAD - N)))
    srcp = jnp.concatenate([src, padidx]).reshape(EPAD // W, W)
    dstp = jnp.concatenate([dst, padidx]).reshape(EPAD // W, W)

    xa = jnp.concatenate(
        [x, jnp.ones((N, 1), f32), jnp.zeros((N, C1 - DIN - 1), f32)], axis=1)
    xa = jnp.concatenate([xa, jnp.zeros((NPAD - N, C1), f32)], axis=0)
    x_pad = jnp.concatenate([x, jnp.zeros((NPAD - N, DIN), f32)], axis=0)

    zer1 = jnp.zeros((NPAD, C1), f32)
    zer = jnp.zeros((NPAD, DIN), f32)

    s1 = _agg_l1(xa, srcp, dstp, zer1).reshape(2, NPAD, C1)
    h1s_and_recip = _tc_layer1(s1, x_pad, Wl1, bl1.reshape(1, DH), Wr1)
    h1s, recipb = h1s_and_recip[:4], h1s_and_recip[4]

    s2s = _agg_l2(*h1s, srcp, dstp, zer)
    p3, r3 = _tc_layer2(s2s, h1s, recipb, Wl2, bl2.reshape(1, DH), Wr2,
                        Wl3, Wr3)

    s3 = _agg_l3(p3, srcp, dstp, zer).reshape(2, NPAD, DOUT)
    batchf = jnp.concatenate(
        [batch.astype(f32), jnp.full((NPAD - N,), float(G), f32)]
    ).reshape(NBLK, 1, R)
    out = _tc_final(s3, recipb, r3, batchf, bl3.reshape(1, DOUT),
                    ln_g.reshape(1, DOUT), ln_b.reshape(1, DOUT))
    return out
